# Initial kernel scaffold; baseline (speedup 1.0000x reference)
import functools

import jax
import jax.numpy as jnp
from jax import lax
from jax.experimental import pallas as pl
from jax.experimental.pallas import tpu as pltpu
from jax.experimental.pallas import tpu_sc as plsc

N_COMP = 16
OUT_CH = 32
H = 128
W = 128
N_PTS = 131072
C = N_COMP * OUT_CH

NC = 2
NS = 16
NW = NC * NS
L = 16

P = 16
PTS_PER_W = N_PTS // NW
ROUNDS = PTS_PER_W // P


def _f32(v):
    return jnp.full((L,), v, dtype=jnp.float32)


def _splat(p):
    return jnp.full((L,), p, dtype=jnp.int32)


def _sc_body(xt_hbm, uv_tab, st_tab, out_hbm,
             xs_v, idx_u, idx_s, w_v, rows_u, rows_s, out_v, sem):
    wid = lax.axis_index("s") * NC + lax.axis_index("c")
    w_base = wid * PTS_PER_W

    def round_body(r, carry):
        base = w_base + r * P

        pltpu.sync_copy(xt_hbm.at[:, pl.ds(base, P)], xs_v)

        def plane_prep(row_x, row_y, idx_ref, w_off):
            gx = xs_v[row_x, :] * (W - 1.0)
            gy = xs_v[row_y, :] * (H - 1.0)
            xi = gx.astype(jnp.int32)
            yi = gy.astype(jnp.int32)
            xi = jnp.minimum(jnp.maximum(xi, 0), W - 2)
            yi = jnp.minimum(jnp.maximum(yi, 0), H - 2)
            fx = gx - xi.astype(jnp.float32)
            fy = gy - yi.astype(jnp.float32)
            ib = yi * W + xi
            idx_ref[pl.ds(0, L)] = ib
            idx_ref[pl.ds(P, L)] = ib + 1
            idx_ref[pl.ds(2 * P, L)] = ib + W
            idx_ref[pl.ds(3 * P, L)] = ib + W + 1
            wx0 = 1.0 - fx
            wy0 = 1.0 - fy
            w_v[w_off + 0, :] = wx0 * wy0
            w_v[w_off + 1, :] = fx * wy0
            w_v[w_off + 2, :] = wx0 * fy
            w_v[w_off + 3, :] = fx * fy

        plane_prep(0, 1, idx_u, 0)
        plane_prep(2, 3, idx_s, 4)

        cp_u = pltpu.async_copy(uv_tab.at[idx_u], rows_u, sem)
        cp_s = pltpu.async_copy(st_tab.at[idx_s], rows_s, sem)
        cp_u.wait()
        cp_s.wait()

        def point_body(p, carry2):
            wts = [plsc.load_gather(w_v, [_splat(j), _splat(p)])
                   for j in range(8)]
            acc0 = _f32(0.0)
            acc1 = _f32(0.0)
            for k in range(N_COMP):
                c0 = k * OUT_CH
                u0 = (wts[0] * rows_u[p, pl.ds(c0, L)]
                      + wts[1] * rows_u[P + p, pl.ds(c0, L)]
                      + wts[2] * rows_u[2 * P + p, pl.ds(c0, L)]
                      + wts[3] * rows_u[3 * P + p, pl.ds(c0, L)])
                s0 = (wts[4] * rows_s[p, pl.ds(c0, L)]
                      + wts[5] * rows_s[P + p, pl.ds(c0, L)]
                      + wts[6] * rows_s[2 * P + p, pl.ds(c0, L)]
                      + wts[7] * rows_s[3 * P + p, pl.ds(c0, L)])
                acc0 = acc0 + u0 * s0
                c1 = c0 + L
                u1 = (wts[0] * rows_u[p, pl.ds(c1, L)]
                      + wts[1] * rows_u[P + p, pl.ds(c1, L)]
                      + wts[2] * rows_u[2 * P + p, pl.ds(c1, L)]
                      + wts[3] * rows_u[3 * P + p, pl.ds(c1, L)])
                s1 = (wts[4] * rows_s[p, pl.ds(c1, L)]
                      + wts[5] * rows_s[P + p, pl.ds(c1, L)]
                      + wts[6] * rows_s[2 * P + p, pl.ds(c1, L)]
                      + wts[7] * rows_s[3 * P + p, pl.ds(c1, L)])
                acc1 = acc1 + u1 * s1
            out_v[p, pl.ds(0, L)] = jnp.maximum(acc0, 0.0)
            out_v[p, pl.ds(L, L)] = jnp.maximum(acc1, 0.0)
            return carry2

        lax.fori_loop(0, P, point_body, 0, unroll=False)

        pltpu.sync_copy(out_v, out_hbm.at[pl.ds(base, P)])
        return carry

    lax.fori_loop(0, ROUNDS, round_body, 0, unroll=False)


@jax.jit
def _run(xt, uv_tab, st_tab):
    kern = pl.kernel(
        _sc_body,
        out_type=jax.ShapeDtypeStruct((N_PTS, OUT_CH), jnp.float32),
        mesh=plsc.VectorSubcoreMesh(
            core_axis_name="c", subcore_axis_name="s",
            num_cores=NC, num_subcores=NS),
        scratch_types=[
            pltpu.VMEM((4, P), jnp.float32),
            pltpu.VMEM((4 * P,), jnp.int32),
            pltpu.VMEM((4 * P,), jnp.int32),
            pltpu.VMEM((8, P), jnp.float32),
            pltpu.VMEM((4 * P, C), jnp.float32),
            pltpu.VMEM((4 * P, C), jnp.float32),
            pltpu.VMEM((P, OUT_CH), jnp.float32),
            pltpu.SemaphoreType.DMA,
        ],
    )
    return kern(xt, uv_tab, st_tab)


def kernel(x, uv_plane, st_plane):
    uv_tab = jnp.transpose(uv_plane[0], (1, 2, 0)).reshape(H * W, C)
    st_tab = jnp.transpose(st_plane[0], (1, 2, 0)).reshape(H * W, C)
    xt = x.T
    return _run(xt, uv_tab, st_tab)


# speedup vs baseline: 10.1947x; 10.1947x over previous
import functools

import jax
import jax.numpy as jnp
from jax import lax
from jax.experimental import pallas as pl
from jax.experimental.pallas import tpu as pltpu
from jax.experimental.pallas import tpu_sc as plsc

N_COMP = 16
OUT_CH = 32
H = 128
W = 128
N_PTS = 131072
C = N_COMP * OUT_CH

NC = 2
NS = 16
NW = NC * NS
L = 16

P = 16
PTS_PER_W = N_PTS // NW
ROUNDS = PTS_PER_W // P


def _f32(v):
    return jnp.full((L,), v, dtype=jnp.float32)


def _splat(p):
    return jnp.full((L,), p, dtype=jnp.int32)


_GDN = lax.GatherDimensionNumbers(
    offset_dims=(), collapsed_slice_dims=(0,), start_index_map=(0,))


def _bcast_lane(w, sp):
    return lax.gather(w, sp[:, None], _GDN, (1,),
                      mode=lax.GatherScatterMode.PROMISE_IN_BOUNDS)


def _sc_body(x0_hbm, x1_hbm, x2_hbm, x3_hbm, uv_tab, st_tab, out_hbm,
             xs_v, idx_u, idx_s, rows_u, rows_s, out_v, sem):
    wid = lax.axis_index("s") * NC + lax.axis_index("c")
    w_base = wid * PTS_PER_W

    def round_body(r, carry):
        base = w_base + r * P

        pltpu.sync_copy(x0_hbm.at[pl.ds(base, P)], xs_v.at[pl.ds(0, P)])
        pltpu.sync_copy(x1_hbm.at[pl.ds(base, P)], xs_v.at[pl.ds(P, P)])
        pltpu.sync_copy(x2_hbm.at[pl.ds(base, P)], xs_v.at[pl.ds(2 * P, P)])
        pltpu.sync_copy(x3_hbm.at[pl.ds(base, P)], xs_v.at[pl.ds(3 * P, P)])

        def plane_prep(row_x, row_y, idx_ref):
            gx = xs_v[pl.ds(row_x * P, L)] * (W - 1.0)
            gy = xs_v[pl.ds(row_y * P, L)] * (H - 1.0)
            xi = gx.astype(jnp.int32)
            yi = gy.astype(jnp.int32)
            xi = jnp.minimum(jnp.maximum(xi, 0), W - 2)
            yi = jnp.minimum(jnp.maximum(yi, 0), H - 2)
            fx = gx - xi.astype(jnp.float32)
            fy = gy - yi.astype(jnp.float32)
            ib = yi * W + xi
            idx_ref[pl.ds(0, L)] = ib
            idx_ref[pl.ds(P, L)] = ib + 1
            idx_ref[pl.ds(2 * P, L)] = ib + W
            idx_ref[pl.ds(3 * P, L)] = ib + W + 1
            wx0 = 1.0 - fx
            wy0 = 1.0 - fy
            return (wx0 * wy0, fx * wy0, wx0 * fy, fx * fy)

        w_uv = plane_prep(0, 1, idx_u)
        w_st = plane_prep(2, 3, idx_s)
        w_all = w_uv + w_st

        cp_u = pltpu.async_copy(uv_tab.at[idx_u], rows_u, sem)
        cp_s = pltpu.async_copy(st_tab.at[idx_s], rows_s, sem)
        cp_u.wait()
        cp_s.wait()

        def point_body(p, carry2):
            sp = _splat(p)
            wts = [_bcast_lane(w, sp) for w in w_all]
            acc0 = _f32(0.0)
            acc1 = _f32(0.0)
            for k in range(N_COMP):
                c0 = k * OUT_CH
                u0 = (wts[0] * rows_u[p, pl.ds(c0, L)]
                      + wts[1] * rows_u[P + p, pl.ds(c0, L)]
                      + wts[2] * rows_u[2 * P + p, pl.ds(c0, L)]
                      + wts[3] * rows_u[3 * P + p, pl.ds(c0, L)])
                s0 = (wts[4] * rows_s[p, pl.ds(c0, L)]
                      + wts[5] * rows_s[P + p, pl.ds(c0, L)]
                      + wts[6] * rows_s[2 * P + p, pl.ds(c0, L)]
                      + wts[7] * rows_s[3 * P + p, pl.ds(c0, L)])
                acc0 = acc0 + u0 * s0
                c1 = c0 + L
                u1 = (wts[0] * rows_u[p, pl.ds(c1, L)]
                      + wts[1] * rows_u[P + p, pl.ds(c1, L)]
                      + wts[2] * rows_u[2 * P + p, pl.ds(c1, L)]
                      + wts[3] * rows_u[3 * P + p, pl.ds(c1, L)])
                s1 = (wts[4] * rows_s[p, pl.ds(c1, L)]
                      + wts[5] * rows_s[P + p, pl.ds(c1, L)]
                      + wts[6] * rows_s[2 * P + p, pl.ds(c1, L)]
                      + wts[7] * rows_s[3 * P + p, pl.ds(c1, L)])
                acc1 = acc1 + u1 * s1
            out_v[p, pl.ds(0, L)] = jnp.maximum(acc0, 0.0)
            out_v[p, pl.ds(L, L)] = jnp.maximum(acc1, 0.0)
            return carry2

        lax.fori_loop(0, P, point_body, 0, unroll=False)

        pltpu.sync_copy(out_v, out_hbm.at[pl.ds(base, P)])
        return carry

    lax.fori_loop(0, ROUNDS, round_body, 0, unroll=False)


@jax.jit
def _run(x0, x1, x2, x3, uv_tab, st_tab):
    kern = pl.kernel(
        _sc_body,
        out_type=jax.ShapeDtypeStruct((N_PTS, OUT_CH), jnp.float32),
        mesh=plsc.VectorSubcoreMesh(
            core_axis_name="c", subcore_axis_name="s",
            num_cores=NC, num_subcores=NS),
        scratch_types=[
            pltpu.VMEM((4 * P,), jnp.float32),
            pltpu.VMEM((4 * P,), jnp.int32),
            pltpu.VMEM((4 * P,), jnp.int32),
            pltpu.VMEM((4 * P, C), jnp.float32),
            pltpu.VMEM((4 * P, C), jnp.float32),
            pltpu.VMEM((P, OUT_CH), jnp.float32),
            pltpu.SemaphoreType.DMA,
        ],
    )
    return kern(x0, x1, x2, x3, uv_tab, st_tab)


def kernel(x, uv_plane, st_plane):
    uv_tab = jnp.transpose(uv_plane[0], (1, 2, 0)).reshape(H * W, C)
    st_tab = jnp.transpose(st_plane[0], (1, 2, 0)).reshape(H * W, C)
    return _run(x[:, 0], x[:, 1], x[:, 2], x[:, 3], uv_tab, st_tab)


# speedup vs baseline: 15.3590x; 1.5066x over previous
import functools

import jax
import jax.numpy as jnp
from jax import lax
from jax.experimental import pallas as pl
from jax.experimental.pallas import tpu as pltpu
from jax.experimental.pallas import tpu_sc as plsc

N_COMP = 16
OUT_CH = 32
H = 128
W = 128
N_PTS = 131072
C = N_COMP * OUT_CH

NC = 2
NS = 16
NW = NC * NS
L = 16

P = 16
PTS_PER_W = N_PTS // NW
ROUNDS = PTS_PER_W // P
RPF = 4
SR = ROUNDS // RPF


def _f32(v):
    return jnp.full((L,), v, dtype=jnp.float32)


def _splat(p):
    return jnp.full((L,), p, dtype=jnp.int32)


_GDN = lax.GatherDimensionNumbers(
    offset_dims=(), collapsed_slice_dims=(0,), start_index_map=(0,))


def _bcast_lane(w, sp):
    return lax.gather(w, sp[:, None], _GDN, (1,),
                      mode=lax.GatherScatterMode.PROMISE_IN_BOUNDS)


def _unpack(v):
    lo = lax.bitcast_convert_type(lax.shift_left(v, 16), jnp.float32)
    hi = lax.bitcast_convert_type(jnp.bitwise_and(v, -65536), jnp.float32)
    return lo, hi


def _sc_body(x0_hbm, x1_hbm, x2_hbm, x3_hbm, uv_tab, st_tab, out_hbm,
             xs_v, idx_u, idx_s, rows_u, rows_s, out_v, sem):
    wid = lax.axis_index("s") * NC + lax.axis_index("c")
    w_base = wid * PTS_PER_W

    pltpu.sync_copy(x0_hbm.at[pl.ds(w_base, PTS_PER_W)],
                    xs_v.at[pl.ds(0, PTS_PER_W)])
    pltpu.sync_copy(x1_hbm.at[pl.ds(w_base, PTS_PER_W)],
                    xs_v.at[pl.ds(PTS_PER_W, PTS_PER_W)])
    pltpu.sync_copy(x2_hbm.at[pl.ds(w_base, PTS_PER_W)],
                    xs_v.at[pl.ds(2 * PTS_PER_W, PTS_PER_W)])
    pltpu.sync_copy(x3_hbm.at[pl.ds(w_base, PTS_PER_W)],
                    xs_v.at[pl.ds(3 * PTS_PER_W, PTS_PER_W)])

    def plane_prep(r, row_x, row_y, idx_ref):
        gx = xs_v[pl.ds(row_x * PTS_PER_W + r * P, L)] * (W - 1.0)
        gy = xs_v[pl.ds(row_y * PTS_PER_W + r * P, L)] * (H - 1.0)
        xi = gx.astype(jnp.int32)
        yi = gy.astype(jnp.int32)
        xi = jnp.minimum(jnp.maximum(xi, 0), W - 2)
        yi = jnp.minimum(jnp.maximum(yi, 0), H - 2)
        fx = gx - xi.astype(jnp.float32)
        fy = gy - yi.astype(jnp.float32)
        ib = yi * W + xi
        idx_ref[pl.ds(0, L)] = ib
        idx_ref[pl.ds(P, L)] = ib + 1
        idx_ref[pl.ds(2 * P, L)] = ib + W
        idx_ref[pl.ds(3 * P, L)] = ib + W + 1
        wx0 = 1.0 - fx
        wy0 = 1.0 - fy
        return (wx0 * wy0, fx * wy0, wx0 * fy, fx * fy)

    def sr_body(sr, carry):
        for b in range(RPF):
            r = sr * RPF + b
            w_uv = plane_prep(r, 0, 1, idx_u)
            w_st = plane_prep(r, 2, 3, idx_s)
            w_all = w_uv + w_st

            cp_u = pltpu.async_copy(uv_tab.at[idx_u], rows_u, sem)
            cp_s = pltpu.async_copy(st_tab.at[idx_s], rows_s, sem)
            cp_u.wait()
            cp_s.wait()

            def point_body(p, carry2, w_all=w_all, ob=b * P):
                sp = _splat(p)
                wts = [_bcast_lane(w, sp) for w in w_all]
                acc0 = _f32(0.0)
                acc1 = _f32(0.0)
                for k in range(N_COMP):
                    c0 = k * L
                    a00 = _unpack(rows_u[p, pl.ds(c0, L)])
                    a01 = _unpack(rows_u[P + p, pl.ds(c0, L)])
                    a10 = _unpack(rows_u[2 * P + p, pl.ds(c0, L)])
                    a11 = _unpack(rows_u[3 * P + p, pl.ds(c0, L)])
                    b00 = _unpack(rows_s[p, pl.ds(c0, L)])
                    b01 = _unpack(rows_s[P + p, pl.ds(c0, L)])
                    b10 = _unpack(rows_s[2 * P + p, pl.ds(c0, L)])
                    b11 = _unpack(rows_s[3 * P + p, pl.ds(c0, L)])
                    u0 = (wts[0] * a00[0] + wts[1] * a01[0]
                          + wts[2] * a10[0] + wts[3] * a11[0])
                    s0 = (wts[4] * b00[0] + wts[5] * b01[0]
                          + wts[6] * b10[0] + wts[7] * b11[0])
                    acc0 = acc0 + u0 * s0
                    u1 = (wts[0] * a00[1] + wts[1] * a01[1]
                          + wts[2] * a10[1] + wts[3] * a11[1])
                    s1 = (wts[4] * b00[1] + wts[5] * b01[1]
                          + wts[6] * b10[1] + wts[7] * b11[1])
                    acc1 = acc1 + u1 * s1
                out_v[ob + p, pl.ds(0, L)] = jnp.maximum(acc0, 0.0)
                out_v[ob + p, pl.ds(L, L)] = jnp.maximum(acc1, 0.0)
                return carry2

            lax.fori_loop(0, P, point_body, 0, unroll=False)

        pltpu.sync_copy(out_v, out_hbm.at[pl.ds(w_base + sr * RPF * P,
                                                RPF * P)])
        return carry

    lax.fori_loop(0, SR, sr_body, 0, unroll=False)


@jax.jit
def _run(x0, x1, x2, x3, uv_tab, st_tab):
    kern = pl.kernel(
        _sc_body,
        out_type=jax.ShapeDtypeStruct((N_PTS, OUT_CH), jnp.float32),
        mesh=plsc.VectorSubcoreMesh(
            core_axis_name="c", subcore_axis_name="s",
            num_cores=NC, num_subcores=NS),
        scratch_types=[
            pltpu.VMEM((4 * PTS_PER_W,), jnp.float32),
            pltpu.VMEM((4 * P,), jnp.int32),
            pltpu.VMEM((4 * P,), jnp.int32),
            pltpu.VMEM((4 * P, C // 2), jnp.int32),
            pltpu.VMEM((4 * P, C // 2), jnp.int32),
            pltpu.VMEM((RPF * P, OUT_CH), jnp.float32),
            pltpu.SemaphoreType.DMA,
        ],
    )
    return kern(x0, x1, x2, x3, uv_tab, st_tab)


def _prep_table(plane):
    t = jnp.transpose(plane[0], (1, 2, 0)).reshape(H * W, C)
    t = t.reshape(H * W, N_COMP, 2, L).transpose(0, 1, 3, 2)
    t = t.astype(jnp.bfloat16).reshape(H * W, C // 2, 2)
    return lax.bitcast_convert_type(t, jnp.int32)


def kernel(x, uv_plane, st_plane):
    return _run(x[:, 0], x[:, 1], x[:, 2], x[:, 3],
                _prep_table(uv_plane), _prep_table(st_plane))


# speedup vs baseline: 23.1912x; 1.5099x over previous
import functools

import jax
import jax.numpy as jnp
from jax import lax
from jax.experimental import pallas as pl
from jax.experimental.pallas import tpu as pltpu
from jax.experimental.pallas import tpu_sc as plsc

N_COMP = 16
OUT_CH = 32
H = 128
W = 128
N_PTS = 131072
C = N_COMP * OUT_CH

NC = 2
NS = 16
NW = NC * NS
L = 16

P = 16
PTS_PER_W = N_PTS // NW
ROUNDS = PTS_PER_W // P
RPF = 4
SR = ROUNDS // RPF


def _f32(v):
    return jnp.full((L,), v, dtype=jnp.float32)


def _splat(p):
    return jnp.full((L,), p, dtype=jnp.int32)


_GDN = lax.GatherDimensionNumbers(
    offset_dims=(), collapsed_slice_dims=(0,), start_index_map=(0,))


def _bcast_lane(w, sp):
    return lax.gather(w, sp[:, None], _GDN, (1,),
                      mode=lax.GatherScatterMode.PROMISE_IN_BOUNDS)


def _unpack(v):
    lo = lax.bitcast_convert_type(lax.shift_left(v, 16), jnp.float32)
    hi = lax.bitcast_convert_type(jnp.bitwise_and(v, -65536), jnp.float32)
    return lo, hi


def _sc_body(x0_hbm, x1_hbm, x2_hbm, x3_hbm, uv_tab, st_tab, out_hbm,
             xs_v, idx_u0, idx_s0, idx_u1, idx_s1,
             rows_u0, rows_s0, rows_u1, rows_s1, out_v, sem0, sem1):
    wid = lax.axis_index("s") * NC + lax.axis_index("c")
    w_base = wid * PTS_PER_W

    pltpu.sync_copy(x0_hbm.at[pl.ds(w_base, PTS_PER_W)],
                    xs_v.at[pl.ds(0, PTS_PER_W)])
    pltpu.sync_copy(x1_hbm.at[pl.ds(w_base, PTS_PER_W)],
                    xs_v.at[pl.ds(PTS_PER_W, PTS_PER_W)])
    pltpu.sync_copy(x2_hbm.at[pl.ds(w_base, PTS_PER_W)],
                    xs_v.at[pl.ds(2 * PTS_PER_W, PTS_PER_W)])
    pltpu.sync_copy(x3_hbm.at[pl.ds(w_base, PTS_PER_W)],
                    xs_v.at[pl.ds(3 * PTS_PER_W, PTS_PER_W)])

    def plane_prep(r, row_x, row_y, idx_ref):
        gx = xs_v[pl.ds(row_x * PTS_PER_W + r * P, L)] * (W - 1.0)
        gy = xs_v[pl.ds(row_y * PTS_PER_W + r * P, L)] * (H - 1.0)
        xi = gx.astype(jnp.int32)
        yi = gy.astype(jnp.int32)
        xi = jnp.minimum(jnp.maximum(xi, 0), W - 2)
        yi = jnp.minimum(jnp.maximum(yi, 0), H - 2)
        fx = gx - xi.astype(jnp.float32)
        fy = gy - yi.astype(jnp.float32)
        ib = yi * W + xi
        idx_ref[pl.ds(0, L)] = ib
        idx_ref[pl.ds(P, L)] = ib + 1
        idx_ref[pl.ds(2 * P, L)] = ib + W
        idx_ref[pl.ds(3 * P, L)] = ib + W + 1
        wx0 = 1.0 - fx
        wy0 = 1.0 - fy
        return (wx0 * wy0, fx * wy0, wx0 * fy, fx * fy)

    slots = ((idx_u0, idx_s0, rows_u0, rows_s0, sem0),
             (idx_u1, idx_s1, rows_u1, rows_s1, sem1))

    def fire(r, slot):
        idx_u, idx_s, rows_u, rows_s, sem = slot
        w_uv = plane_prep(r, 0, 1, idx_u)
        w_st = plane_prep(r, 2, 3, idx_s)

        @pl.when(r < ROUNDS)
        def _():
            pltpu.async_copy(uv_tab.at[idx_u], rows_u, sem)
            pltpu.async_copy(st_tab.at[idx_s], rows_s, sem)

        return w_uv + w_st

    w0 = fire(0, slots[0])

    def sr_body(sr, w_carry):
        w_all = w_carry
        for b in range(RPF):
            r = sr * RPF + b
            cur = slots[b % 2]
            w_next = fire(r + 1, slots[(b + 1) % 2])

            idx_u, idx_s, rows_u, rows_s, sem = cur
            pltpu.make_async_copy(uv_tab.at[idx_u], rows_u, sem).wait()
            pltpu.make_async_copy(st_tab.at[idx_s], rows_s, sem).wait()

            def point_body(p, carry2, w_all=w_all, ob=b * P,
                           rows_u=rows_u, rows_s=rows_s):
                sp = _splat(p)
                wts = [_bcast_lane(w, sp) for w in w_all]
                acc0 = _f32(0.0)
                acc1 = _f32(0.0)
                for k in range(N_COMP):
                    c0 = k * L
                    a00 = _unpack(rows_u[p, pl.ds(c0, L)])
                    a01 = _unpack(rows_u[P + p, pl.ds(c0, L)])
                    a10 = _unpack(rows_u[2 * P + p, pl.ds(c0, L)])
                    a11 = _unpack(rows_u[3 * P + p, pl.ds(c0, L)])
                    b00 = _unpack(rows_s[p, pl.ds(c0, L)])
                    b01 = _unpack(rows_s[P + p, pl.ds(c0, L)])
                    b10 = _unpack(rows_s[2 * P + p, pl.ds(c0, L)])
                    b11 = _unpack(rows_s[3 * P + p, pl.ds(c0, L)])
                    u0 = (wts[0] * a00[0] + wts[1] * a01[0]
                          + wts[2] * a10[0] + wts[3] * a11[0])
                    s0 = (wts[4] * b00[0] + wts[5] * b01[0]
                          + wts[6] * b10[0] + wts[7] * b11[0])
                    acc0 = acc0 + u0 * s0
                    u1 = (wts[0] * a00[1] + wts[1] * a01[1]
                          + wts[2] * a10[1] + wts[3] * a11[1])
                    s1 = (wts[4] * b00[1] + wts[5] * b01[1]
                          + wts[6] * b10[1] + wts[7] * b11[1])
                    acc1 = acc1 + u1 * s1
                out_v[ob + p, pl.ds(0, L)] = jnp.maximum(acc0, 0.0)
                out_v[ob + p, pl.ds(L, L)] = jnp.maximum(acc1, 0.0)
                return carry2

            lax.fori_loop(0, P, point_body, 0, unroll=False)
            w_all = w_next

        pltpu.sync_copy(out_v, out_hbm.at[pl.ds(w_base + sr * RPF * P,
                                                RPF * P)])
        return w_all

    lax.fori_loop(0, SR, sr_body, w0, unroll=False)


@jax.jit
def _run(x0, x1, x2, x3, uv_tab, st_tab):
    kern = pl.kernel(
        _sc_body,
        out_type=jax.ShapeDtypeStruct((N_PTS, OUT_CH), jnp.float32),
        mesh=plsc.VectorSubcoreMesh(
            core_axis_name="c", subcore_axis_name="s",
            num_cores=NC, num_subcores=NS),
        scratch_types=[
            pltpu.VMEM((4 * PTS_PER_W + P,), jnp.float32),
            pltpu.VMEM((4 * P,), jnp.int32),
            pltpu.VMEM((4 * P,), jnp.int32),
            pltpu.VMEM((4 * P,), jnp.int32),
            pltpu.VMEM((4 * P,), jnp.int32),
            pltpu.VMEM((4 * P, C // 2), jnp.int32),
            pltpu.VMEM((4 * P, C // 2), jnp.int32),
            pltpu.VMEM((4 * P, C // 2), jnp.int32),
            pltpu.VMEM((4 * P, C // 2), jnp.int32),
            pltpu.VMEM((RPF * P, OUT_CH), jnp.float32),
            pltpu.SemaphoreType.DMA,
            pltpu.SemaphoreType.DMA,
        ],
    )
    return kern(x0, x1, x2, x3, uv_tab, st_tab)


def _prep_table(plane):
    t = jnp.transpose(plane[0], (1, 2, 0)).reshape(H * W, C)
    t = t.reshape(H * W, N_COMP, 2, L).transpose(0, 1, 3, 2)
    t = t.astype(jnp.bfloat16).reshape(H * W, C // 2, 2)
    return lax.bitcast_convert_type(t, jnp.int32)


def kernel(x, uv_plane, st_plane):
    return _run(x[:, 0], x[:, 1], x[:, 2], x[:, 3],
                _prep_table(uv_plane), _prep_table(st_plane))
